# baseline (speedup 1.0000x reference)
import functools

import jax
import jax.numpy as jnp
from jax import lax
from jax.experimental import pallas as pl
from jax.experimental.pallas import tpu as pltpu
from jax.experimental.pallas import tpu_sc as plsc

NC = 2
NS = 16


def kernel(past_k, past_v, past_cu_seqlens, cur_k, cur_v, cur_cu_seqlens):
    nb = past_cu_seqlens.shape[0] - 1
    past_len = past_k.shape[0] // nb
    cur_len = cur_k.shape[0] // nb
    new_len = past_len + cur_len
    tail = past_k.shape[1:]
    total_new = nb * new_len

    workers_per_tensor = NC * NS // 2
    halves = workers_per_tensor // nb
    rows_per_half = past_len // halves

    mesh = plsc.VectorSubcoreMesh(core_axis_name="c", subcore_axis_name="s")

    @functools.partial(
        pl.kernel,
        mesh=mesh,
        out_type=[
            jax.ShapeDtypeStruct((total_new,) + tail, past_k.dtype),
            jax.ShapeDtypeStruct((total_new,) + tail, past_v.dtype),
            jax.ShapeDtypeStruct(past_cu_seqlens.shape, past_cu_seqlens.dtype),
        ],
        scratch_types=[
            pltpu.VMEM((16,), jnp.int32),
            pltpu.VMEM((16,), jnp.int32),
            pltpu.VMEM((16,), jnp.int32),
            pltpu.SemaphoreType.DMA,
            pltpu.SemaphoreType.DMA,
        ],
    )
    def splice(pk, pv, pcu, ck, cv, ccu, nk, nv, ncu, a_v, b_v, o_v, sem0, sem1):
        wid = lax.axis_index("s") * NC + lax.axis_index("c")

        def copy_tensor(w, past_ref, cur_ref, out_ref):
            b = w // halves
            h = w % halves
            src = b * past_len + h * rows_per_half
            dst = b * new_len + h * rows_per_half
            big = pltpu.async_copy(
                past_ref.at[pl.ds(src, rows_per_half)],
                out_ref.at[pl.ds(dst, rows_per_half)],
                sem0,
            )

            @pl.when(h == halves - 1)
            def _():
                pltpu.async_copy(
                    cur_ref.at[pl.ds(b * cur_len, cur_len)],
                    out_ref.at[pl.ds(b * new_len + past_len, cur_len)],
                    sem1,
                ).wait()

            big.wait()

        @pl.when(wid < workers_per_tensor)
        def _():
            copy_tensor(wid, pk, ck, nk)

        @pl.when(wid >= workers_per_tensor)
        def _():
            copy_tensor(wid - workers_per_tensor, pv, cv, nv)

        @pl.when(wid == 0)
        def _():
            n = pcu.shape[0]
            pltpu.sync_copy(pcu, a_v.at[pl.ds(0, n)])
            pltpu.sync_copy(ccu, b_v.at[pl.ds(0, n)])
            o_v[...] = a_v[...] + b_v[...]
            pltpu.sync_copy(o_v.at[pl.ds(0, n)], ncu)

    return tuple(splice(past_k, past_v, past_cu_seqlens, cur_k, cur_v, cur_cu_seqlens))


# speedup vs baseline: 36.1212x; 36.1212x over previous
import functools

import jax
import jax.numpy as jnp
from jax import lax
from jax.experimental import pallas as pl
from jax.experimental.pallas import tpu as pltpu
from jax.experimental.pallas import tpu_sc as plsc

NC = 2
NS = 16
CH = 16


def kernel(past_k, past_v, past_cu_seqlens, cur_k, cur_v, cur_cu_seqlens):
    nb = past_cu_seqlens.shape[0] - 1
    past_len = past_k.shape[0] // nb
    cur_len = cur_k.shape[0] // nb
    new_len = past_len + cur_len
    tail = past_k.shape[1:]
    total_new = nb * new_len

    workers_per_tensor = NC * NS // 2
    halves = workers_per_tensor // nb
    rows_per_half = past_len // halves
    n_ch = rows_per_half // CH

    mesh = plsc.VectorSubcoreMesh(core_axis_name="c", subcore_axis_name="s")

    @functools.partial(
        pl.kernel,
        mesh=mesh,
        out_type=[
            jax.ShapeDtypeStruct((total_new,) + tail, past_k.dtype),
            jax.ShapeDtypeStruct((total_new,) + tail, past_v.dtype),
            jax.ShapeDtypeStruct(past_cu_seqlens.shape, past_cu_seqlens.dtype),
        ],
        scratch_types=[
            pltpu.VMEM((CH,) + tail, past_k.dtype),
            pltpu.VMEM((CH,) + tail, past_k.dtype),
            pltpu.VMEM((16,), jnp.int32),
            pltpu.VMEM((16,), jnp.int32),
            pltpu.VMEM((16,), jnp.int32),
            pltpu.SemaphoreType.DMA,
            pltpu.SemaphoreType.DMA,
            pltpu.SemaphoreType.DMA,
            pltpu.SemaphoreType.DMA,
        ],
    )
    def splice(pk, pv, pcu, ck, cv, ccu, nk, nv, ncu,
               buf0, buf1, a_v, b_v, o_v, g0, g1, s0, s1):
        wid = lax.axis_index("s") * NC + lax.axis_index("c")

        def stream_tensor(w, past_ref, cur_ref, out_ref):
            b = w // halves
            h = w % halves
            src0 = b * past_len + h * rows_per_half
            dst0 = b * new_len + h * rows_per_half

            def gather(ci, buf, sem):
                pltpu.make_async_copy(
                    past_ref.at[pl.ds(src0 + ci * CH, CH)], buf, sem).start()

            def scatter(ci, buf, sem):
                pltpu.make_async_copy(
                    buf, out_ref.at[pl.ds(dst0 + ci * CH, CH)], sem).start()

            gather(0, buf0, g0)
            gather(1, buf1, g1)

            def body(i, carry):
                for j, (buf, g, s) in enumerate(((buf0, g0, s0), (buf1, g1, s1))):
                    ci = i * 2 + j
                    pltpu.make_async_copy(
                        past_ref.at[pl.ds(src0, CH)], buf, g).wait()
                    scatter(ci, buf, s)
                    pltpu.make_async_copy(
                        buf, out_ref.at[pl.ds(dst0, CH)], s).wait()

                    @pl.when(ci + 2 < n_ch)
                    def _():
                        gather(ci + 2, buf, g)
                return carry

            lax.fori_loop(0, n_ch // 2, body, 0)

            @pl.when(h == halves - 1)
            def _():
                pltpu.sync_copy(cur_ref.at[pl.ds(b * cur_len, cur_len)],
                                buf0.at[pl.ds(0, cur_len)])
                pltpu.sync_copy(buf0.at[pl.ds(0, cur_len)],
                                out_ref.at[pl.ds(b * new_len + past_len, cur_len)])

        @pl.when(wid < workers_per_tensor)
        def _():
            stream_tensor(wid, pk, ck, nk)

        @pl.when(wid >= workers_per_tensor)
        def _():
            stream_tensor(wid - workers_per_tensor, pv, cv, nv)

        @pl.when(wid == 0)
        def _():
            n = pcu.shape[0]
            pltpu.sync_copy(pcu, a_v.at[pl.ds(0, n)])
            pltpu.sync_copy(ccu, b_v.at[pl.ds(0, n)])
            o_v[...] = a_v[...] + b_v[...]
            pltpu.sync_copy(o_v.at[pl.ds(0, n)], ncu)

    return tuple(splice(past_k, past_v, past_cu_seqlens, cur_k, cur_v, cur_cu_seqlens))
